# baseline (speedup 1.0000x reference)
import jax
import jax.numpy as jnp
from jax.experimental import pallas as pl
from jax.experimental.pallas import tpu as pltpu


def _matmul_body(x_ref, wt_ref, o_ref):
    o_ref[...] = jnp.dot(x_ref[...], wt_ref[...],
                         preferred_element_type=jnp.float32)


def _matmul(x, Wt):
    M, K = x.shape
    N = Wt.shape[1]
    BM = 1000
    return pl.pallas_call(
        _matmul_body,
        grid=(M // BM,),
        in_specs=[
            pl.BlockSpec((BM, K), lambda i: (i, 0)),
            pl.BlockSpec((K, N), lambda i: (0, 0)),
        ],
        out_specs=pl.BlockSpec((BM, N), lambda i: (i, 0)),
        out_shape=jax.ShapeDtypeStruct((M, N), jnp.float32),
    )(x, Wt)


def kernel(x, edge_index, W, b):
    N = x.shape[0]
    edge_index = edge_index.astype(jnp.int32)
    src = edge_index[0]
    dst = edge_index[1]
    loop = jnp.arange(N, dtype=jnp.int32)
    src2 = jnp.concatenate([src, loop], axis=0)
    dst2 = jnp.concatenate([dst, loop], axis=0)
    deg = jnp.zeros((N,), dtype=x.dtype).at[dst2].add(1.0)
    dis = jnp.where(deg > 0, jax.lax.rsqrt(deg), 0.0)
    norm = dis[src2] * dis[dst2]
    h = _matmul(x, W.T)
    msg = h[src2] * norm[:, None]
    out = jnp.zeros((N, W.shape[0]), dtype=x.dtype).at[dst2].add(msg)
    return jax.nn.relu(out + b)
